# baseline (speedup 1.0000x reference)
import jax
import jax.numpy as jnp
from jax.experimental import pallas as pl
from jax.experimental.pallas import tpu as pltpu

B = 8
T = 2048
DIN = 756
K = 1024
D = 32
Q = 8
N = B * T

TILE = 1024


def _vq_kernel(x_ref, win_ref, wout_ref, cbt_ref, cb_ref, cb2_ref,
               out_ref, allq_ref, idx_ref, cbp_scr):
    f32 = jnp.float32
    bf16 = jnp.bfloat16

    @pl.when(pl.program_id(0) == 0)
    def _build_split():
        for q in range(Q):
            cb = cb_ref[q]
            hi = cb.astype(bf16)
            rem1 = cb - hi.astype(f32)
            mid = rem1.astype(bf16)
            lo = (rem1 - mid.astype(f32)).astype(bf16)
            cbp_scr[q] = jnp.concatenate([hi, mid, lo], axis=-1)

    x = x_ref[...]
    z = jnp.dot(x, win_ref[...], preferred_element_type=f32)
    res = z
    qsum = jnp.zeros_like(z)
    idx_cols = []
    for q in range(Q):
        cbt = cbt_ref[q]
        cb2 = cb2_ref[q]
        r2 = jnp.sum(res * res, axis=1, keepdims=True)
        s = jnp.dot(res.astype(bf16), cbt, preferred_element_type=f32)
        dist = (r2 - 2.0 * s) + cb2
        idx = jnp.argmin(dist, axis=1)[:, None]
        iota = jax.lax.broadcasted_iota(jnp.int32, dist.shape, 1)
        onehot = (iota == idx).astype(bf16)
        parts = jnp.dot(onehot, cbp_scr[q], preferred_element_type=f32)
        quant = (parts[:, :D] + parts[:, D:2 * D]) + parts[:, 2 * D:]
        res = res - quant
        qsum = qsum + quant
        allq_ref[q] = quant
        idx_cols.append(idx)
    idx_ref[...] = jnp.concatenate(idx_cols, axis=1)
    out = jnp.dot(qsum.astype(bf16), wout_ref[...], preferred_element_type=f32)
    out_ref[...] = jnp.maximum(out, 0.0)


def kernel(inputs, W_in, W_out, codebooks):
    bf16 = jnp.bfloat16
    x = inputs.reshape(N, DIN).astype(bf16)
    w_in_perm = W_in.reshape(3, 252, D).transpose(1, 0, 2).reshape(DIN, D).astype(bf16)
    cbt = codebooks.transpose(0, 2, 1).astype(bf16)
    cb2 = jnp.sum(codebooks ** 2, axis=-1)[:, None, :]

    grid = (N // TILE,)
    out, allq, idx = pl.pallas_call(
        _vq_kernel,
        grid=grid,
        in_specs=[
            pl.BlockSpec((TILE, DIN), lambda i: (i, 0)),
            pl.BlockSpec((DIN, D), lambda i: (0, 0)),
            pl.BlockSpec((D, DIN), lambda i: (0, 0)),
            pl.BlockSpec((Q, D, K), lambda i: (0, 0, 0)),
            pl.BlockSpec((Q, K, D), lambda i: (0, 0, 0)),
            pl.BlockSpec((Q, 1, K), lambda i: (0, 0, 0)),
        ],
        out_specs=[
            pl.BlockSpec((TILE, DIN), lambda i: (i, 0)),
            pl.BlockSpec((Q, TILE, D), lambda i: (0, i, 0)),
            pl.BlockSpec((TILE, Q), lambda i: (i, 0)),
        ],
        out_shape=[
            jax.ShapeDtypeStruct((N, DIN), jnp.float32),
            jax.ShapeDtypeStruct((Q, N, D), jnp.float32),
            jax.ShapeDtypeStruct((N, Q), jnp.int32),
        ],
        scratch_shapes=[pltpu.VMEM((Q, K, 3 * D), jnp.bfloat16)],
    )(x, w_in_perm, W_out.astype(bf16), cbt, codebooks, cb2)

    return (out.reshape(B, T, DIN),
            allq.reshape(Q, B, T, D),
            idx.reshape(B, T, Q))


# speedup vs baseline: 1.1357x; 1.1357x over previous
import jax
import jax.numpy as jnp
from jax.experimental import pallas as pl
from jax.experimental.pallas import tpu as pltpu

B = 8
T = 2048
DIN = 756
K = 1024
D = 32
Q = 8
N = B * T

TILE = 1024


def _vq_kernel(z_ref, wout_ref, cbt_ref, cb_ref, cb2_ref,
               out_ref, allq_ref, idx_ref, cbp_scr):
    f32 = jnp.float32
    bf16 = jnp.bfloat16

    @pl.when(pl.program_id(0) == 0)
    def _build_split():
        for q in range(Q):
            cb = cb_ref[q]
            hi = cb.astype(bf16)
            rem1 = cb - hi.astype(f32)
            mid = rem1.astype(bf16)
            lo = (rem1 - mid.astype(f32)).astype(bf16)
            cbp_scr[q] = jnp.concatenate([hi, mid, lo], axis=-1)

    res = z_ref[...]
    qsum = jnp.zeros_like(res)
    idx_cols = []
    for q in range(Q):
        cbt = cbt_ref[q]
        cb2 = cb2_ref[q]
        r2 = jnp.sum(res * res, axis=1, keepdims=True)
        s = jnp.dot(res, cbt, preferred_element_type=f32)
        dist = (r2 - 2.0 * s) + cb2
        minv = jnp.min(dist, axis=1, keepdims=True)
        iota = jax.lax.broadcasted_iota(jnp.int32, dist.shape, 1)
        idx = jnp.min(jnp.where(dist == minv, iota, K), axis=1, keepdims=True)
        onehot = (iota == idx).astype(bf16)
        parts = jnp.dot(onehot, cbp_scr[q], preferred_element_type=f32)
        quant = (parts[:, :D] + parts[:, D:2 * D]) + parts[:, 2 * D:]
        res = res - quant
        qsum = qsum + quant
        allq_ref[q] = quant
        idx_cols.append(idx)
    idx_ref[...] = jnp.concatenate(idx_cols, axis=1)
    out = jnp.dot(qsum, wout_ref[...], preferred_element_type=f32)
    out_ref[...] = jnp.maximum(out, 0.0)


def kernel(inputs, W_in, W_out, codebooks):
    x = jnp.transpose(inputs, (0, 1, 3, 2)).reshape(B, -1, DIN)
    z = jnp.einsum('btd,de->bte', x, W_in).reshape(N, D)
    cbt = codebooks.transpose(0, 2, 1)
    cb2 = jnp.sum(codebooks ** 2, axis=-1)[:, None, :]

    grid = (N // TILE,)
    out, allq, idx = pl.pallas_call(
        _vq_kernel,
        grid=grid,
        in_specs=[
            pl.BlockSpec((TILE, D), lambda i: (i, 0)),
            pl.BlockSpec((D, DIN), lambda i: (0, 0)),
            pl.BlockSpec((Q, D, K), lambda i: (0, 0, 0)),
            pl.BlockSpec((Q, K, D), lambda i: (0, 0, 0)),
            pl.BlockSpec((Q, 1, K), lambda i: (0, 0, 0)),
        ],
        out_specs=[
            pl.BlockSpec((TILE, DIN), lambda i: (i, 0)),
            pl.BlockSpec((Q, TILE, D), lambda i: (0, i, 0)),
            pl.BlockSpec((TILE, Q), lambda i: (i, 0)),
        ],
        out_shape=[
            jax.ShapeDtypeStruct((N, DIN), jnp.float32),
            jax.ShapeDtypeStruct((Q, N, D), jnp.float32),
            jax.ShapeDtypeStruct((N, Q), jnp.int32),
        ],
        scratch_shapes=[pltpu.VMEM((Q, K, 3 * D), jnp.bfloat16)],
    )(z, W_out, cbt, codebooks, cb2)

    return (out.reshape(B, T, DIN),
            allq.reshape(Q, B, T, D),
            idx.reshape(B, T, Q))


# speedup vs baseline: 1.2100x; 1.0654x over previous
import jax
import jax.numpy as jnp
from jax.experimental import pallas as pl
from jax.experimental.pallas import tpu as pltpu

B = 8
T = 2048
DIN = 756
K = 1024
D = 32
Q = 8
N = B * T

TILE = 1024
W = 3 * D + 3


def _vq_kernel(z_ref, wout_ref, cbt_ref, cb_ref, cb2_ref,
               out_ref, allq_ref, idx_ref, cbp_scr, q_scr, i_scr):
    f32 = jnp.float32
    bf16 = jnp.bfloat16

    @pl.when(pl.program_id(0) == 0)
    def _build_split():
        kio = jax.lax.broadcasted_iota(jnp.int32, (K, 1), 0)
        k_hi = (kio & ~31).astype(bf16)
        k_lo = (kio & 31).astype(bf16)
        ones = jnp.ones((K, 1), bf16)
        for q in range(Q):
            cb = cb_ref[q]
            hi = cb.astype(bf16)
            rem1 = cb - hi.astype(f32)
            mid = rem1.astype(bf16)
            lo = (rem1 - mid.astype(f32)).astype(bf16)
            cbp_scr[q] = jnp.concatenate([hi, mid, lo, k_hi, k_lo, ones],
                                         axis=-1)

    res = z_ref[...]
    qsum = jnp.zeros_like(res)
    idx_cols = []
    for q in range(Q):
        cbt = cbt_ref[q]
        cb2 = cb2_ref[q]
        r2 = jnp.sum(res * res, axis=1, keepdims=True)
        s = jnp.dot(res, cbt, preferred_element_type=f32)
        dist = (r2 - 2.0 * s) + cb2
        minv = jnp.min(dist, axis=1, keepdims=True)
        m = (dist == minv).astype(bf16)
        parts = jnp.dot(m, cbp_scr[q], preferred_element_type=f32)
        quant = (parts[:, :D] + parts[:, D:2 * D]) + parts[:, 2 * D:3 * D]
        idxf = parts[:, 3 * D:3 * D + 1] + parts[:, 3 * D + 1:3 * D + 2]
        count = parts[:, 3 * D + 2:3 * D + 3]
        q_scr[...] = quant
        i_scr[...] = idxf.astype(jnp.int32)

        @pl.when(jnp.max(count) > 1.0)
        def _tie_fallback():
            iota = jax.lax.broadcasted_iota(jnp.int32, dist.shape, 1)
            idx_e = jnp.min(jnp.where(dist == minv, iota, K), axis=1,
                            keepdims=True)
            onehot = (iota == idx_e).astype(bf16)
            pe = jnp.dot(onehot, cbp_scr[q], preferred_element_type=f32)
            q_scr[...] = (pe[:, :D] + pe[:, D:2 * D]) + pe[:, 2 * D:3 * D]
            i_scr[...] = idx_e

        quant = q_scr[...]
        res = res - quant
        qsum = qsum + quant
        allq_ref[q] = quant
        idx_cols.append(i_scr[...])
    idx_ref[...] = jnp.concatenate(idx_cols, axis=1)
    out = jnp.dot(qsum, wout_ref[...], preferred_element_type=f32)
    out_ref[...] = jnp.maximum(out, 0.0)


def kernel(inputs, W_in, W_out, codebooks):
    x = jnp.transpose(inputs, (0, 1, 3, 2)).reshape(B, -1, DIN)
    z = jnp.einsum('btd,de->bte', x, W_in).reshape(N, D)
    cbt = codebooks.transpose(0, 2, 1)
    cb2 = jnp.sum(codebooks ** 2, axis=-1)[:, None, :]

    grid = (N // TILE,)
    out, allq, idx = pl.pallas_call(
        _vq_kernel,
        grid=grid,
        in_specs=[
            pl.BlockSpec((TILE, D), lambda i: (i, 0)),
            pl.BlockSpec((D, DIN), lambda i: (0, 0)),
            pl.BlockSpec((Q, D, K), lambda i: (0, 0, 0)),
            pl.BlockSpec((Q, K, D), lambda i: (0, 0, 0)),
            pl.BlockSpec((Q, 1, K), lambda i: (0, 0, 0)),
        ],
        out_specs=[
            pl.BlockSpec((TILE, DIN), lambda i: (i, 0)),
            pl.BlockSpec((Q, TILE, D), lambda i: (0, i, 0)),
            pl.BlockSpec((TILE, Q), lambda i: (i, 0)),
        ],
        out_shape=[
            jax.ShapeDtypeStruct((N, DIN), jnp.float32),
            jax.ShapeDtypeStruct((Q, N, D), jnp.float32),
            jax.ShapeDtypeStruct((N, Q), jnp.int32),
        ],
        scratch_shapes=[pltpu.VMEM((Q, K, W), jnp.bfloat16),
                        pltpu.VMEM((TILE, D), jnp.float32),
                        pltpu.VMEM((TILE, 1), jnp.int32)],
    )(z, W_out, cbt, codebooks, cb2)

    return (out.reshape(B, T, DIN),
            allq.reshape(Q, B, T, D),
            idx.reshape(B, T, Q))
